# baseline (speedup 1.0000x reference)
import functools

import jax
import jax.numpy as jnp
from jax import lax
from jax.experimental import pallas as pl
from jax.experimental.pallas import tpu as pltpu
from jax.experimental.pallas import tpu_sc as plsc

_VOCAB = 1_000_000
_BATCH = 16384
_FIELDS = 26
_DIM = 32
_NC = 2
_NS = 16
_NW = _NC * _NS
_BPW = _BATCH // _NW
_NBUF = 4

_TB = 64
_NBLK = _VOCAB // _TB
_KMAX = -(-_NBLK // _NW)
_NIN = 4

_mesh = plsc.VectorSubcoreMesh(
    core_axis_name="c", subcore_axis_name="s", num_cores=_NC, num_subcores=_NS
)


@functools.partial(
    pl.kernel,
    mesh=_mesh,
    out_type=jax.ShapeDtypeStruct((_VOCAB, _DIM), jnp.float32),
    scratch_types=[
        pltpu.VMEM((_NIN, _DIM, _TB), jnp.float32),
        pltpu.VMEM((2, _TB, _DIM), jnp.float32),
        pltpu.SemaphoreType.DMA,
        pltpu.SemaphoreType.DMA,
    ],
    compiler_params=pltpu.CompilerParams(
        use_tc_tiling_on_sc=False, needs_layout_passes=False
    ),
)
def _format_kernel(tcol_hbm, trm_hbm, bufs, tbufs, sem_in, sem_out):
    wid = lax.axis_index("s") * _NC + lax.axis_index("c")
    nk = jnp.where(wid < _NBLK - _NW * (_KMAX - 1), _KMAX, _KMAX - 1)

    lanes = lax.iota(jnp.int32, 16)

    def blk(k):
        return (wid + _NW * k) * _TB

    for p in range(_NIN - 1):
        @pl.when(p < nk)
        def _():
            pltpu.async_copy(
                tcol_hbm.at[:, pl.ds(blk(p), _TB)], bufs.at[p], sem_in
            )

    @pl.loop(0, _KMAX)
    def _step(k):
        @pl.when(k < nk)
        def _():
            b = lax.rem(k, _NIN)
            t = lax.rem(k, 2)
            pltpu.make_async_copy(
                tcol_hbm.at[:, pl.ds(0, _TB)], bufs.at[b], sem_in
            ).wait()

            @pl.when(k + _NIN - 1 < nk)
            def _():
                nxt = k + _NIN - 1
                pltpu.async_copy(
                    tcol_hbm.at[:, pl.ds(blk(nxt), _TB)],
                    bufs.at[lax.rem(nxt, _NIN)],
                    sem_in,
                )

            @pl.when(k >= 2)
            def _():
                pltpu.make_async_copy(
                    tbufs.at[0], trm_hbm.at[pl.ds(0, _TB)], sem_out
                ).wait()

            for v in range(_TB * _DIM // 16):
                r, c0 = (v * 16) // _DIM, (v * 16) % _DIM
                vals = plsc.load_gather(
                    bufs.at[b], [c0 + lanes, jnp.full((16,), r, jnp.int32)]
                )
                tbufs[t, r, pl.ds(c0, 16)] = vals

            pltpu.async_copy(tbufs.at[t], trm_hbm.at[pl.ds(blk(k), _TB)], sem_out)

    for _ in range(2):
        pltpu.make_async_copy(
            tbufs.at[0], trm_hbm.at[pl.ds(0, _TB)], sem_out
        ).wait()


@functools.partial(
    pl.kernel,
    mesh=_mesh,
    out_type=jax.ShapeDtypeStruct((_BATCH, _FIELDS, _DIM), jnp.float32),
    scratch_types=[
        pltpu.VMEM((_FIELDS, _BPW), jnp.int32),
        pltpu.VMEM((_NBUF, _BPW, _DIM), jnp.float32),
        pltpu.SemaphoreType.DMA,
    ],
    compiler_params=pltpu.CompilerParams(use_tc_tiling_on_sc=False),
)
def _gather_kernel(table_hbm, idxt_hbm, out_hbm, idx_v, rows_v, sem):
    wid = lax.axis_index("s") * _NC + lax.axis_index("c")
    b0 = wid * _BPW
    pltpu.sync_copy(idxt_hbm.at[:, pl.ds(b0, _BPW)], idx_v)

    for f in range(_NBUF - 1):
        pltpu.async_copy(table_hbm.at[idx_v.at[f]], rows_v.at[f], sem)

    @pl.loop(0, _FIELDS)
    def _field(f):
        b = lax.rem(f, _NBUF)
        pltpu.make_async_copy(table_hbm.at[idx_v.at[f]], rows_v.at[b], sem).wait()

        pltpu.sync_copy(rows_v.at[b], out_hbm.at[pl.ds(b0, _BPW), f])

        @pl.when(f + _NBUF - 1 < _FIELDS)
        def _():
            nxt = f + _NBUF - 1
            pltpu.async_copy(
                table_hbm.at[idx_v.at[nxt]], rows_v.at[lax.rem(nxt, _NBUF)], sem
            )


def kernel(x, table):
    table_rm = _format_kernel(table.T)
    return _gather_kernel(table_rm, x.T.astype(jnp.int32))


# speedup vs baseline: 3.6099x; 3.6099x over previous
import functools

import jax
import jax.numpy as jnp
from jax import lax
from jax.experimental import pallas as pl
from jax.experimental.pallas import tpu as pltpu
from jax.experimental.pallas import tpu_sc as plsc

_BATCH = 16384
_FIELDS = 26
_DIM = 32
_NC = 2
_NS = 16
_NW = _NC * _NS
_BPW = _BATCH // _NW
_NBUF = 4

_mesh = plsc.VectorSubcoreMesh(
    core_axis_name="c", subcore_axis_name="s", num_cores=_NC, num_subcores=_NS
)


@functools.partial(
    pl.kernel,
    mesh=_mesh,
    out_type=jax.ShapeDtypeStruct((_BATCH, _FIELDS, _DIM), jnp.bfloat16),
    scratch_types=[
        pltpu.VMEM((_FIELDS, _BPW), jnp.int32),
        pltpu.VMEM((_NBUF, _BPW, _DIM), jnp.bfloat16),
        pltpu.SemaphoreType.DMA,
    ],
    compiler_params=pltpu.CompilerParams(use_tc_tiling_on_sc=False),
)
def _gather_kernel(table_hbm, idxt_hbm, out_hbm, idx_v, rows_v, sem):
    wid = lax.axis_index("s") * _NC + lax.axis_index("c")
    b0 = wid * _BPW
    pltpu.sync_copy(idxt_hbm.at[:, pl.ds(b0, _BPW)], idx_v)

    for f in range(_NBUF - 1):
        pltpu.async_copy(table_hbm.at[idx_v.at[f]], rows_v.at[f], sem)

    @pl.loop(0, _FIELDS)
    def _field(f):
        b = lax.rem(f, _NBUF)
        pltpu.make_async_copy(table_hbm.at[idx_v.at[f]], rows_v.at[b], sem).wait()

        pltpu.sync_copy(rows_v.at[b], out_hbm.at[pl.ds(b0, _BPW), f])

        @pl.when(f + _NBUF - 1 < _FIELDS)
        def _():
            nxt = f + _NBUF - 1
            pltpu.async_copy(
                table_hbm.at[idx_v.at[nxt]], rows_v.at[lax.rem(nxt, _NBUF)], sem
            )


def kernel(x, table):
    out = _gather_kernel(table.astype(jnp.bfloat16), x.T.astype(jnp.int32))
    return out.astype(jnp.float32)


# speedup vs baseline: 4.5677x; 1.2653x over previous
import functools

import jax
import jax.numpy as jnp
from jax import lax
from jax.experimental import pallas as pl
from jax.experimental.pallas import tpu as pltpu
from jax.experimental.pallas import tpu_sc as plsc

_BATCH = 16384
_FIELDS = 26
_DIM = 32
_NC = 2
_NS = 16
_NW = _NC * _NS
_BPW = _BATCH // _NW
_NBUF = 4

_mesh = plsc.VectorSubcoreMesh(
    core_axis_name="c", subcore_axis_name="s", num_cores=_NC, num_subcores=_NS
)


@functools.partial(
    pl.kernel,
    mesh=_mesh,
    out_type=jax.ShapeDtypeStruct((_BATCH, _FIELDS, _DIM), jnp.float32),
    scratch_types=[
        pltpu.VMEM((_FIELDS, _BPW), jnp.int32),
        pltpu.VMEM((_NBUF, _BPW, _DIM), jnp.float32),
        pltpu.SemaphoreType.DMA,
    ],
    compiler_params=pltpu.CompilerParams(use_tc_tiling_on_sc=False),
)
def _gather_kernel(table_hbm, idxt_hbm, out_hbm, idx_v, rows_v, sem):
    wid = lax.axis_index("s") * _NC + lax.axis_index("c")
    b0 = wid * _BPW
    pltpu.sync_copy(idxt_hbm.at[:, pl.ds(b0, _BPW)], idx_v)

    for f in range(_NBUF - 1):
        pltpu.async_copy(table_hbm.at[idx_v.at[f]], rows_v.at[f], sem)

    @pl.loop(0, _FIELDS)
    def _field(f):
        b = lax.rem(f, _NBUF)
        pltpu.make_async_copy(table_hbm.at[idx_v.at[f]], rows_v.at[b], sem).wait()

        pltpu.sync_copy(rows_v.at[b], out_hbm.at[pl.ds(b0, _BPW), f])

        @pl.when(f + _NBUF - 1 < _FIELDS)
        def _():
            nxt = f + _NBUF - 1
            pltpu.async_copy(
                table_hbm.at[idx_v.at[nxt]], rows_v.at[lax.rem(nxt, _NBUF)], sem
            )


def kernel(x, table):
    return _gather_kernel(table, x.T.astype(jnp.int32))
